# Initial kernel scaffold; baseline (speedup 1.0000x reference)
import jax
import jax.numpy as jnp
from jax.experimental import pallas as pl


def kernel(prediction, target):
    raise NotImplementedError("write your pallas kernel here")


# speedup vs baseline: 12.5400x; 12.5400x over previous
import jax
import jax.numpy as jnp
from jax.experimental import pallas as pl
from jax.experimental.pallas import tpu as pltpu

_ROWS = 8192
_COLS = 2048
_BLOCK_ROWS = 512


def _sum_body(x_ref, o_ref):
    @pl.when(pl.program_id(0) == 0)
    def _init():
        o_ref[0, 0] = 0.0

    o_ref[0, 0] += jnp.sum(x_ref[...])


def kernel(prediction, target):
    del target
    x = prediction.reshape(_ROWS, _COLS)
    out = pl.pallas_call(
        _sum_body,
        grid=(_ROWS // _BLOCK_ROWS,),
        in_specs=[pl.BlockSpec((_BLOCK_ROWS, _COLS), lambda i: (i, 0))],
        out_specs=pl.BlockSpec(memory_space=pltpu.SMEM),
        out_shape=jax.ShapeDtypeStruct((1, 1), jnp.float32),
    )(x)
    return out[0, 0]


# speedup vs baseline: 23.9087x; 1.9066x over previous
import jax
import jax.numpy as jnp
from jax.experimental import pallas as pl
from jax.experimental.pallas import tpu as pltpu


def _sum_body(x_ref, o_ref):
    @pl.when((pl.program_id(0) == 0) & (pl.program_id(1) == 0))
    def _init():
        o_ref[0, 0] = 0.0

    o_ref[0, 0] += jnp.sum(x_ref[...])


def kernel(prediction, target):
    del target
    B, F, H, W = prediction.shape
    out = pl.pallas_call(
        _sum_body,
        grid=(B, F),
        in_specs=[pl.BlockSpec((1, 1, H, W), lambda i, j: (i, j, 0, 0))],
        out_specs=pl.BlockSpec(memory_space=pltpu.SMEM),
        out_shape=jax.ShapeDtypeStruct((1, 1), jnp.float32),
    )(prediction)
    return out[0, 0]


# speedup vs baseline: 42.3778x; 1.7725x over previous
import jax
import jax.numpy as jnp
from jax.experimental import pallas as pl
from jax.experimental.pallas import tpu as pltpu

_FB = 2


def _sum_body(x_ref, o_ref, acc_ref):
    i = pl.program_id(0)
    j = pl.program_id(1)

    @pl.when((i == 0) & (j == 0))
    def _init():
        acc_ref[...] = jnp.zeros_like(acc_ref)

    x = x_ref[...].reshape(-1, 8, 512)
    acc_ref[...] += jnp.sum(x, axis=0)

    @pl.when((i == pl.num_programs(0) - 1) & (j == pl.num_programs(1) - 1))
    def _fini():
        o_ref[0, 0] = jnp.sum(acc_ref[...])


def kernel(prediction, target):
    del target
    B, F, H, W = prediction.shape
    out = pl.pallas_call(
        _sum_body,
        grid=(B, F // _FB),
        in_specs=[pl.BlockSpec((1, _FB, H, W), lambda i, j: (i, j, 0, 0))],
        out_specs=pl.BlockSpec(memory_space=pltpu.SMEM),
        out_shape=jax.ShapeDtypeStruct((1, 1), jnp.float32),
        scratch_shapes=[pltpu.VMEM((8, 512), jnp.float32)],
    )(prediction)
    return out[0, 0]


# speedup vs baseline: 56.9071x; 1.3429x over previous
import jax
import jax.numpy as jnp
from jax.experimental import pallas as pl
from jax.experimental.pallas import tpu as pltpu

_FB = 4


def _sum_body(x_ref, o_ref, acc_ref):
    i = pl.program_id(0)
    j = pl.program_id(1)

    @pl.when((i == 0) & (j == 0))
    def _init():
        acc_ref[...] = jnp.zeros_like(acc_ref)

    x = x_ref[...].reshape(-1, 8, 512)
    acc_ref[...] += jnp.sum(x, axis=0)

    @pl.when((i == pl.num_programs(0) - 1) & (j == pl.num_programs(1) - 1))
    def _fini():
        o_ref[0, 0] = jnp.sum(acc_ref[...])


def kernel(prediction, target):
    del target
    B, F, H, W = prediction.shape
    out = pl.pallas_call(
        _sum_body,
        grid=(B, F // _FB),
        in_specs=[pl.BlockSpec((1, _FB, H, W), lambda i, j: (i, j, 0, 0))],
        out_specs=pl.BlockSpec(memory_space=pltpu.SMEM),
        out_shape=jax.ShapeDtypeStruct((1, 1), jnp.float32),
        scratch_shapes=[pltpu.VMEM((8, 512), jnp.float32)],
    )(prediction)
    return out[0, 0]


# speedup vs baseline: 62.0084x; 1.0896x over previous
import jax
import jax.numpy as jnp
from jax.experimental import pallas as pl
from jax.experimental.pallas import tpu as pltpu

_FB = 8


def _sum_body(x_ref, o_ref, acc_ref):
    i = pl.program_id(0)
    j = pl.program_id(1)

    @pl.when((i == 0) & (j == 0))
    def _init():
        acc_ref[...] = jnp.zeros_like(acc_ref)

    x = x_ref[...].reshape(-1, 8, 512)
    acc_ref[...] += jnp.sum(x, axis=0)

    @pl.when((i == pl.num_programs(0) - 1) & (j == pl.num_programs(1) - 1))
    def _fini():
        o_ref[0, 0] = jnp.sum(acc_ref[...])


def kernel(prediction, target):
    del target
    B, F, H, W = prediction.shape
    out = pl.pallas_call(
        _sum_body,
        grid=(B, F // _FB),
        in_specs=[pl.BlockSpec((1, _FB, H, W), lambda i, j: (i, j, 0, 0))],
        out_specs=pl.BlockSpec(memory_space=pltpu.SMEM),
        out_shape=jax.ShapeDtypeStruct((1, 1), jnp.float32),
        scratch_shapes=[pltpu.VMEM((8, 512), jnp.float32)],
    )(prediction)
    return out[0, 0]
